# baseline (speedup 1.0000x reference)
import functools

import jax
import jax.numpy as jnp
from jax import lax
from jax.experimental import pallas as pl
from jax.experimental.pallas import tpu as pltpu
from jax.experimental.pallas import tpu_sc as plsc

B, L = 4096, 50
N = B * L
D_W, D_F, D_OUT = 32, 16, 64
NC, NS = 2, 16
NW = NC * NS
ROWS_PER_W = N // NW
CHUNK = 128
NCHUNK = ROWS_PER_W // CHUNK
NGRP = CHUNK // 16

_mesh = plsc.VectorSubcoreMesh(
    core_axis_name="c", subcore_axis_name="s", num_cores=NC, num_subcores=NS
)


@functools.partial(
    pl.kernel,
    out_type=jax.ShapeDtypeStruct((N * D_OUT,), jnp.float32),
    mesh=_mesh,
    compiler_params=pltpu.CompilerParams(use_tc_tiling_on_sc=True,
                                         needs_layout_passes=False),
    scratch_types=[
        pltpu.VMEM((ROWS_PER_W,), jnp.int32),
        pltpu.VMEM((ROWS_PER_W,), jnp.int32),
        pltpu.VMEM((ROWS_PER_W,), jnp.int32),
        pltpu.VMEM((ROWS_PER_W,), jnp.int32),
        pltpu.VMEM((16000,), jnp.float32),
        pltpu.VMEM((16000,), jnp.float32),
        pltpu.VMEM((CHUNK, 128), jnp.float32),
        pltpu.VMEM((CHUNK * D_OUT,), jnp.float32),
        pltpu.SemaphoreType.DMA,
    ],
)
def _emb_kernel(xq, xo, lo, ro, wv, pf1, pf2, out, qv, ov, lv, rv,
                pf1v, pf2v, wbuf, obuf, sem):
    wid = lax.axis_index("s") * NC + lax.axis_index("c")
    base = wid * ROWS_PER_W
    rows = pl.ds(base, ROWS_PER_W)
    pltpu.sync_copy(xq.at[rows], qv)
    pltpu.sync_copy(xo.at[rows], ov)
    pltpu.sync_copy(lo.at[rows], lv)
    pltpu.sync_copy(ro.at[rows], rv)
    pltpu.sync_copy(pf1, pf1v)
    pltpu.sync_copy(pf2, pf2v)
    iota = lax.iota(jnp.int32, 16)

    def chunk_body(ci, carry):
        c0 = ci * CHUNK
        pltpu.async_copy(wv.at[qv.at[pl.ds(c0, CHUNK)]], wbuf, sem).wait()

        def grp_body(g, carry2):
            i0 = g * 16
            offv = ov[pl.ds(c0 + i0, 16)]
            lofv = lv[pl.ds(c0 + i0, 16)]
            rofv = rv[pl.ds(c0 + i0, 16)]
            for k in range(16):
                i = i0 + k
                ri = jnp.full((16,), i, jnp.int32)
                cw = offv[k] + iota
                g0 = plsc.load_gather(wbuf, [ri, cw])
                g1 = plsc.load_gather(wbuf, [ri, cw + 16])
                gl = plsc.load_gather(pf1v, [lofv[k] + iota])
                gr = plsc.load_gather(pf2v, [rofv[k] + iota])
                obuf[pl.ds(i * D_OUT, 16)] = g0
                obuf[pl.ds(i * D_OUT + 16, 16)] = g1
                obuf[pl.ds(i * D_OUT + 32, 16)] = gl
                obuf[pl.ds(i * D_OUT + 48, 16)] = gr
            return carry2

        lax.fori_loop(0, NGRP, grp_body, 0)
        pltpu.sync_copy(obuf, out.at[pl.ds((base + c0) * D_OUT, CHUNK * D_OUT)])
        return carry

    lax.fori_loop(0, NCHUNK, chunk_body, 0)


def kernel(x, ldist, rdist, Wv, pf1, pf2):
    xi = x.reshape(-1).astype(jnp.int32)
    li = ldist.reshape(-1).astype(jnp.int32)
    ri = rdist.reshape(-1).astype(jnp.int32)
    xq = xi >> 2
    xo = (xi & 3) * D_W
    lo = li * D_F
    ro = ri * D_F
    out = _emb_kernel(xq, xo, lo, ro, Wv.reshape(250000, 128),
                      pf1.reshape(-1), pf2.reshape(-1))
    return out.reshape(B, 1, L, D_OUT)
